# Initial kernel scaffold; baseline (speedup 1.0000x reference)
import jax
import jax.numpy as jnp
from jax.experimental import pallas as pl


def kernel(x, table):
    raise NotImplementedError("write your pallas kernel here")


# speedup vs baseline: 1.8456x; 1.8456x over previous
import functools

import jax
import jax.numpy as jnp
from jax import lax
from jax.experimental import pallas as pl
from jax.experimental.pallas import tpu as pltpu
from jax.experimental.pallas import tpu_sc as plsc

VOCAB = 1000000
EMBED_DIM = 64
BATCH = 16384
HIST = 50

B = BATCH * HIST
IDXW = 128
NC, NS = 2, 16
NW = NC * NS
ROWS_PER_W = B // NW
CHUNK = 1024
IROWS = CHUNK // IDXW
NCHUNK = ROWS_PER_W // CHUNK
XROWS_PER_W = ROWS_PER_W // IDXW


@functools.partial(
    pl.kernel,
    mesh=plsc.VectorSubcoreMesh(core_axis_name="c", subcore_axis_name="s"),
    out_type=jax.ShapeDtypeStruct((B, EMBED_DIM), jnp.float32),
    scratch_types=[
        pltpu.VMEM((IROWS, IDXW), jnp.int32),
        pltpu.VMEM((CHUNK, EMBED_DIM), jnp.float32),
        pltpu.SemaphoreType.DMA,
    ],
    compiler_params=pltpu.CompilerParams(use_tc_tiling_on_sc=False),
)
def _gather_kernel(x_hbm, table_hbm, out_hbm, idx_v, rows_v, sem):
    wid = lax.axis_index("s") * NC + lax.axis_index("c")

    def chunk_body(i, carry):
        xrow0 = wid * XROWS_PER_W + i * IROWS
        pltpu.sync_copy(x_hbm.at[pl.ds(xrow0, IROWS)], idx_v)
        descs = [
            pltpu.async_copy(
                table_hbm.at[idx_v.at[j]],
                rows_v.at[pl.ds(j * IDXW, IDXW)],
                sem,
            )
            for j in range(IROWS)
        ]
        for d in descs:
            d.wait()
        out0 = wid * ROWS_PER_W + i * CHUNK
        pltpu.sync_copy(rows_v, out_hbm.at[pl.ds(out0, CHUNK)])
        return carry

    lax.fori_loop(0, NCHUNK, chunk_body, 0)


def kernel(x, table):
    x2d = x.reshape(B // IDXW, IDXW)
    out = _gather_kernel(x2d, table)
    return out.reshape(BATCH, HIST, EMBED_DIM)


# speedup vs baseline: 1.8722x; 1.0144x over previous
import functools

import jax
import jax.numpy as jnp
from jax import lax
from jax.experimental import pallas as pl
from jax.experimental.pallas import tpu as pltpu
from jax.experimental.pallas import tpu_sc as plsc

VOCAB = 1000000
EMBED_DIM = 64
BATCH = 16384
HIST = 50

B = BATCH * HIST
IDXW = 128
NC, NS = 2, 16
NW = NC * NS
ROWS_PER_W = B // NW
CHUNK = 512
IROWS = CHUNK // IDXW
NCHUNK = ROWS_PER_W // CHUNK
XROWS_PER_W = ROWS_PER_W // IDXW
NBUF = 3


@functools.partial(
    pl.kernel,
    mesh=plsc.VectorSubcoreMesh(core_axis_name="c", subcore_axis_name="s"),
    out_type=jax.ShapeDtypeStruct((B, EMBED_DIM), jnp.float32),
    scratch_types=[
        pltpu.VMEM((XROWS_PER_W, IDXW), jnp.int32),
        pltpu.VMEM((CHUNK, EMBED_DIM), jnp.float32),
        pltpu.VMEM((CHUNK, EMBED_DIM), jnp.float32),
        pltpu.VMEM((CHUNK, EMBED_DIM), jnp.float32),
        pltpu.SemaphoreType.DMA,
        pltpu.SemaphoreType.DMA,
        pltpu.SemaphoreType.DMA,
        pltpu.SemaphoreType.DMA,
        pltpu.SemaphoreType.DMA,
        pltpu.SemaphoreType.DMA,
    ],
    compiler_params=pltpu.CompilerParams(use_tc_tiling_on_sc=False),
)
def _gather_kernel(x_hbm, table_hbm, out_hbm, idx_v,
                   rows0, rows1, rows2, g0, g1, g2, s0, s1, s2):
    rows = (rows0, rows1, rows2)
    gsem = (g0, g1, g2)
    ssem = (s0, s1, s2)
    wid = lax.axis_index("s") * NC + lax.axis_index("c")
    obase = wid * ROWS_PER_W

    pltpu.sync_copy(x_hbm.at[pl.ds(wid * XROWS_PER_W, XROWS_PER_W)], idx_v)

    def fire(b, c):
        for j in range(IROWS):
            pltpu.async_copy(
                table_hbm.at[idx_v.at[c * IROWS + j]],
                rows[b].at[pl.ds(j * IDXW, IDXW)],
                gsem[b],
            )

    def drain_store(b, c):
        pltpu.make_async_copy(
            out_hbm.at[pl.ds(obase, CHUNK)], rows[b], gsem[b]
        ).wait()
        pltpu.async_copy(rows[b], out_hbm.at[pl.ds(obase + c * CHUNK, CHUNK)],
                         ssem[b])

    def wait_store(b):
        pltpu.make_async_copy(
            rows[b], out_hbm.at[pl.ds(obase, CHUNK)], ssem[b]
        ).wait()

    fire(0, 0)
    fire(1, 1)
    drain_store(0, 0)
    fire(2, 2)
    drain_store(1, 1)
    wait_store(0)
    fire(0, 3)
    drain_store(2, 2)
    wait_store(1)
    fire(1, 4)

    def body(j, carry):
        c = 3 * j + 3
        for t in range(3):
            ct = c + t
            drain_store(t, ct)
            wait_store((t + 2) % 3)
            fire((t + 2) % 3, ct + 2)
        return carry

    lax.fori_loop(0, 15, body, 0)

    drain_store(0, 48)
    wait_store(2)
    drain_store(1, 49)
    wait_store(0)
    wait_store(1)


def kernel(x, table):
    x2d = x.reshape(B // IDXW, IDXW)
    out = _gather_kernel(x2d, table)
    return out.reshape(BATCH, HIST, EMBED_DIM)


# speedup vs baseline: 1.8844x; 1.0065x over previous
import functools

import jax
import jax.numpy as jnp
from jax import lax
from jax.experimental import pallas as pl
from jax.experimental.pallas import tpu as pltpu
from jax.experimental.pallas import tpu_sc as plsc

VOCAB = 1000000
EMBED_DIM = 64
BATCH = 16384
HIST = 50

NC, NS = 2, 16
NW = NC * NS
B_PER_W = BATCH // NW
CB = 8
NCHUNK = B_PER_W // CB
NBUF = 3


@functools.partial(
    pl.kernel,
    mesh=plsc.VectorSubcoreMesh(core_axis_name="c", subcore_axis_name="s"),
    out_type=jax.ShapeDtypeStruct((BATCH, HIST, EMBED_DIM), jnp.float32),
    scratch_types=[
        pltpu.VMEM((B_PER_W, HIST), jnp.int32),
        pltpu.VMEM((CB, HIST, EMBED_DIM), jnp.float32),
        pltpu.VMEM((CB, HIST, EMBED_DIM), jnp.float32),
        pltpu.VMEM((CB, HIST, EMBED_DIM), jnp.float32),
        pltpu.SemaphoreType.DMA,
        pltpu.SemaphoreType.DMA,
        pltpu.SemaphoreType.DMA,
        pltpu.SemaphoreType.DMA,
        pltpu.SemaphoreType.DMA,
        pltpu.SemaphoreType.DMA,
    ],
    compiler_params=pltpu.CompilerParams(use_tc_tiling_on_sc=False),
)
def _gather_kernel(x_hbm, table_hbm, out_hbm, idx_v,
                   rows0, rows1, rows2, g0, g1, g2, s0, s1, s2):
    rows = (rows0, rows1, rows2)
    gsem = (g0, g1, g2)
    ssem = (s0, s1, s2)
    wid = lax.axis_index("s") * NC + lax.axis_index("c")
    bbase = wid * B_PER_W

    pltpu.sync_copy(x_hbm.at[pl.ds(bbase, B_PER_W)], idx_v)

    def fire(b, c):
        for t in range(CB):
            pltpu.async_copy(
                table_hbm.at[idx_v.at[c * CB + t]],
                rows[b].at[t],
                gsem[b],
            )

    def drain_store(b, c):
        pltpu.make_async_copy(
            out_hbm.at[pl.ds(bbase, CB)], rows[b], gsem[b]
        ).wait()
        pltpu.async_copy(rows[b], out_hbm.at[pl.ds(bbase + c * CB, CB)],
                         ssem[b])

    def wait_store(b):
        pltpu.make_async_copy(
            rows[b], out_hbm.at[pl.ds(bbase, CB)], ssem[b]
        ).wait()

    fire(0, 0)
    fire(1, 1)
    drain_store(0, 0)
    fire(2, 2)
    drain_store(1, 1)
    wait_store(0)
    fire(0, 3)

    def body(j, carry):
        c = 3 * j + 2
        for t in range(3):
            ct = c + t
            bt = (2 + t) % 3
            drain_store(bt, ct)
            wait_store((bt + 2) % 3)
            fire((bt + 2) % 3, ct + 2)
        return carry

    lax.fori_loop(0, 20, body, 0)

    drain_store(2, 62)
    wait_store(1)
    drain_store(0, 63)
    wait_store(2)
    wait_store(0)


def kernel(x, table):
    return _gather_kernel(x, table)
